# baseline (speedup 1.0000x reference)
import functools

import jax
import jax.numpy as jnp
from jax import lax
from jax.experimental import pallas as pl
from jax.experimental.pallas import tpu as pltpu
from jax.experimental.pallas import tpu_sc as plsc

B = 16384
D = 784
NC, NS = 2, 16
NW = NC * NS
BPW = B // NW
CH = 64
NCHUNK = BPW // CH


def _gather_body(idx_hbm, tab_hbm, out_hbm, tab_s, idx_v, buf0, buf1, sem0, sem1):
    sid = lax.axis_index("s")
    wid = sid * NC + lax.axis_index("c")
    base = wid * BPW
    @pl.when(sid == 0)
    def _():
        pltpu.sync_copy(tab_hbm, tab_s)

    pltpu.sync_copy(idx_hbm.at[wid], idx_v)
    plsc.subcore_barrier()

    bufs = (buf0, buf1)
    sems = (sem0, sem1)
    pltpu.async_copy(tab_s.at[idx_v.at[0]], bufs[0], sems[0])
    for c in range(NCHUNK):
        buf = bufs[c % 2]
        sem = sems[c % 2]
        pltpu.make_async_copy(tab_s.at[idx_v.at[c]], buf, sem).wait()
        if c + 1 < NCHUNK:
            pltpu.async_copy(
                tab_s.at[idx_v.at[c + 1]], bufs[(c + 1) % 2], sems[(c + 1) % 2]
            )
        pltpu.sync_copy(buf, out_hbm.at[pl.ds(base + c * CH, CH)])


@jax.jit
def _gather(x, logits):
    mesh = plsc.VectorSubcoreMesh(core_axis_name="c", subcore_axis_name="s")
    idx = x.astype(jnp.int32).reshape(NW, NCHUNK, CH)
    run = pl.kernel(
        _gather_body,
        mesh=mesh,
        out_type=jax.ShapeDtypeStruct((B, D), jnp.float32),
        scratch_types=[
            pltpu.VMEM_SHARED((10, D), jnp.float32),
            pltpu.VMEM((NCHUNK, CH), jnp.int32),
            pltpu.VMEM((CH, D), jnp.float32),
            pltpu.VMEM((CH, D), jnp.float32),
            pltpu.SemaphoreType.DMA,
            pltpu.SemaphoreType.DMA,
        ],
        compiler_params=pltpu.CompilerParams(use_tc_tiling_on_sc=False),
    )
    return run(idx, logits)


def kernel(x, logits):
    out = _gather(x, logits)
    return out.reshape(B, 1, 28, 28)


# speedup vs baseline: 3.1903x; 3.1903x over previous
import jax
import jax.numpy as jnp
from jax import lax
from jax.experimental import pallas as pl
from jax.experimental.pallas import tpu as pltpu
from jax.experimental.pallas import tpu_sc as plsc

B = 16384
D = 784
NC, NS = 2, 16
NW = NC * NS
BCOL = B // NW
NG = BCOL // 16
RC = 2
NCHUNK = 28 // RC


def _body(idx_hbm, tabt_hbm, out_hbm, tabt_v, idx_v, buf0, buf1, sem0, sem1):
    wid = lax.axis_index("s") * NC + lax.axis_index("c")
    base = wid * BCOL
    pltpu.sync_copy(tabt_hbm, tabt_v)
    pltpu.sync_copy(idx_hbm.at[pl.ds(base, BCOL)], idx_v)

    bufs = (buf0, buf1)
    sems = (sem0, sem1)
    dnums = lax.GatherDimensionNumbers(
        offset_dims=(), collapsed_slice_dims=(0,), start_index_map=(0,)
    )

    def chunk(c, buf):
        r0 = c * RC

        def per_g8(g8, _):
            xs = [idx_v[pl.ds((g8 * 8 + k) * 16, 16)] for k in range(8)]

            def per_row(r, _):
                for cc in range(28):
                    rowv = tabt_v[pl.ds((r * 28 + cc) * 10, 16)]
                    for k in range(8):
                        v = lax.gather(
                            rowv, xs[k][:, None], dnums, (1,),
                            mode=lax.GatherScatterMode.PROMISE_IN_BOUNDS,
                        )
                        buf[r - r0, cc, pl.ds((g8 * 8 + k) * 16, 16)] = v
                return 0

            lax.fori_loop(r0, r0 + RC, per_row, 0)
            return 0

        lax.fori_loop(0, NG // 8, per_g8, 0)

    def out_slice(c):
        return out_hbm.at[0, pl.ds(c * RC, RC), :, pl.ds(base, BCOL)]

    def two_chunks(h, _):
        c0 = 2 * h
        c1 = 2 * h + 1

        @pl.when(h > 0)
        def _():
            pltpu.make_async_copy(buf0, out_slice(c0 - 2), sem0).wait()

        chunk(c0, buf0)
        pltpu.async_copy(buf0, out_slice(c0), sem0)

        @pl.when(h > 0)
        def _():
            pltpu.make_async_copy(buf1, out_slice(c1 - 2), sem1).wait()

        chunk(c1, buf1)
        pltpu.async_copy(buf1, out_slice(c1), sem1)
        return 0

    lax.fori_loop(0, NCHUNK // 2, two_chunks, 0)
    pltpu.make_async_copy(buf0, out_slice(NCHUNK - 2), sem0).wait()
    pltpu.make_async_copy(buf1, out_slice(NCHUNK - 1), sem1).wait()


@jax.jit
def _gather_t(x, logits):
    mesh = plsc.VectorSubcoreMesh(core_axis_name="c", subcore_axis_name="s")
    idx = x.astype(jnp.int32)
    tabt = jnp.concatenate([logits.T.reshape(D * 10), jnp.zeros((16,), jnp.float32)])
    run = pl.kernel(
        _body,
        mesh=mesh,
        out_type=jax.ShapeDtypeStruct((1, 28, 28, B), jnp.float32),
        scratch_types=[
            pltpu.VMEM((D * 10 + 16,), jnp.float32),
            pltpu.VMEM((BCOL,), jnp.int32),
            pltpu.VMEM((RC, 28, BCOL), jnp.float32),
            pltpu.VMEM((RC, 28, BCOL), jnp.float32),
            pltpu.SemaphoreType.DMA,
            pltpu.SemaphoreType.DMA,
        ],
    )
    out_t = run(idx, tabt)
    return jnp.transpose(out_t, (3, 0, 1, 2))


def kernel(x, logits):
    return _gather_t(x, logits)
